# Initial kernel scaffold; baseline (speedup 1.0000x reference)
import jax
import jax.numpy as jnp
from jax.experimental import pallas as pl


def kernel(hidden_states, W1, b1, W2, b2, router_weight, router_bias, Wf1, bf1, Wf2, bf2):
    raise NotImplementedError("write your pallas kernel here")


# speedup vs baseline: 1.6051x; 1.6051x over previous
import functools

import jax
import jax.numpy as jnp
from jax.experimental import pallas as pl
from jax.experimental.pallas import tpu as pltpu

B, S, D = 2, 2048, 2048
DFF = 4 * D
BS = B * S
MIN_CAP, MAX_CAP = 0.25, 1.0

_T = 256
_F = 512
_NT = BS // _T
_NF = DFF // _F


def _gelu_exact(x):
    return 0.5 * x * (1.0 + jax.lax.erf(x * (2.0 ** -0.5)))


def _head_kernel(hid_ref, w1_ref, b1_ref, w2_ref, b2_ref, rw_ref, rb_ref,
                 w_ref, k_ref):
    hid = hid_ref[...]
    pooled = jnp.mean(hid, axis=1)
    h1 = jnp.dot(pooled.astype(jnp.bfloat16), w1_ref[...].astype(jnp.bfloat16),
                 preferred_element_type=jnp.float32) + b1_ref[...]
    h1 = _gelu_exact(h1)
    c = jax.nn.sigmoid(jnp.dot(h1.astype(jnp.bfloat16),
                               w2_ref[...].astype(jnp.bfloat16),
                               preferred_element_type=jnp.float32) + b2_ref[...])
    cap = MIN_CAP + jnp.mean(c) * (MAX_CAP - MIN_CAP)
    k = (cap * S).astype(jnp.int32)
    k_ref[...] = jnp.reshape(k, (1, 1))

    logits = jnp.dot(jnp.reshape(hid, (BS, D)).astype(jnp.bfloat16),
                     jnp.reshape(rw_ref[...], (D, 1)).astype(jnp.bfloat16),
                     preferred_element_type=jnp.float32)
    logits = jnp.reshape(logits, (B, S)) + rb_ref[0, 0]

    keys = jax.lax.bitcast_convert_type(logits, jnp.int32)
    keys = jnp.where(keys >= 0, keys, keys ^ jnp.int32(0x7FFFFFFF))
    thr = jnp.full((B, 1), -2147483647 - 1, jnp.int32)
    for bit in range(30, -1, -1):
        cand = thr + jnp.int32(1 << bit)
        cnt = jnp.sum((keys >= cand).astype(jnp.int32), axis=1, keepdims=True)
        thr = jnp.where(cnt >= k, cand, thr)
    mask = keys >= thr
    w_ref[...] = jnp.where(mask, jax.nn.sigmoid(logits), 0.0)


def _ffn_kernel(x_ref, w_ref, wf1_ref, bf1_ref, wf2_ref, bf2_ref,
                out_ref, acc_ref):
    f = pl.program_id(1)
    x = x_ref[...]
    h = jnp.dot(x.astype(jnp.bfloat16), wf1_ref[...],
                preferred_element_type=jnp.float32) + bf1_ref[...]
    h = _gelu_exact(h)
    p = jnp.dot(h.astype(jnp.bfloat16), wf2_ref[...],
                preferred_element_type=jnp.float32)

    @pl.when(f == 0)
    def _():
        acc_ref[...] = p

    @pl.when(f > 0)
    def _():
        acc_ref[...] += p

    @pl.when(f == _NF - 1)
    def _():
        out_ref[...] = x + w_ref[...] * (acc_ref[...] + bf2_ref[...])


def kernel(hidden_states, W1, b1, W2, b2, router_weight, router_bias,
           Wf1, bf1, Wf2, bf2):
    w, _k = pl.pallas_call(
        _head_kernel,
        out_shape=[
            jax.ShapeDtypeStruct((B, S), jnp.float32),
            jax.ShapeDtypeStruct((1, 1), jnp.int32),
        ],
    )(hidden_states, W1, b1.reshape(1, D // 4), W2, b2.reshape(1, 1),
      router_weight.reshape(1, D), router_bias.reshape(1, 1))

    out = pl.pallas_call(
        _ffn_kernel,
        grid=(_NT, _NF),
        in_specs=[
            pl.BlockSpec((_T, D), lambda t, f: (t, 0)),
            pl.BlockSpec((_T, 1), lambda t, f: (t, 0)),
            pl.BlockSpec((D, _F), lambda t, f: (0, f)),
            pl.BlockSpec((1, _F), lambda t, f: (0, f)),
            pl.BlockSpec((_F, D), lambda t, f: (f, 0)),
            pl.BlockSpec((1, D), lambda t, f: (0, 0)),
        ],
        out_specs=pl.BlockSpec((_T, D), lambda t, f: (t, 0)),
        out_shape=jax.ShapeDtypeStruct((BS, D), jnp.float32),
        scratch_shapes=[pltpu.VMEM((_T, D), jnp.float32)],
        compiler_params=pltpu.CompilerParams(
            dimension_semantics=("arbitrary", "arbitrary")),
    )(hidden_states.reshape(BS, D), w.reshape(BS, 1),
      Wf1.astype(jnp.bfloat16), bf1.reshape(1, DFF),
      Wf2.astype(jnp.bfloat16), bf2.reshape(1, D))

    return out.reshape(B, S, D)


# speedup vs baseline: 2.2308x; 1.3898x over previous
import functools

import jax
import jax.numpy as jnp
from jax.experimental import pallas as pl
from jax.experimental.pallas import tpu as pltpu

B, S, D = 2, 2048, 2048
DFF = 4 * D
BS = B * S
MIN_CAP, MAX_CAP = 0.25, 1.0

_T = 512
_F = 1024
_NT = BS // _T
_NF = DFF // _F


def _gelu_exact(x):
    return 0.5 * x * (1.0 + jax.lax.erf(x * (2.0 ** -0.5)))


def _head_kernel(hid_ref, w1_ref, b1_ref, w2_ref, b2_ref, rw_ref, rb_ref,
                 w_ref, k_ref):
    hid = hid_ref[...]
    pooled = jnp.mean(hid, axis=1)
    h1 = jnp.dot(pooled.astype(jnp.bfloat16), w1_ref[...].astype(jnp.bfloat16),
                 preferred_element_type=jnp.float32) + b1_ref[...]
    h1 = _gelu_exact(h1)
    c = jax.nn.sigmoid(jnp.dot(h1.astype(jnp.bfloat16),
                               w2_ref[...].astype(jnp.bfloat16),
                               preferred_element_type=jnp.float32) + b2_ref[...])
    cap = MIN_CAP + jnp.mean(c) * (MAX_CAP - MIN_CAP)
    k = (cap * S).astype(jnp.int32)
    k_ref[...] = jnp.reshape(k, (1, 1))

    logits = jnp.dot(jnp.reshape(hid, (BS, D)).astype(jnp.bfloat16),
                     jnp.reshape(rw_ref[...], (D, 1)).astype(jnp.bfloat16),
                     preferred_element_type=jnp.float32)
    logits = jnp.reshape(logits, (B, S)) + rb_ref[0, 0]

    keys = jax.lax.bitcast_convert_type(logits, jnp.int32)
    keys = jnp.where(keys >= 0, keys, keys ^ jnp.int32(0x7FFFFFFF))
    thr = jnp.full((B, 1), -2147483647 - 1, jnp.int32)
    for bit in range(30, -1, -1):
        cand = thr + jnp.int32(1 << bit)
        cnt = jnp.sum((keys >= cand).astype(jnp.int32), axis=1, keepdims=True)
        thr = jnp.where(cnt >= k, cand, thr)
    mask = keys >= thr
    w_ref[...] = jnp.where(mask, jax.nn.sigmoid(logits), 0.0)


def _ffn_kernel(x_ref, w_ref, wf1_ref, bf1_ref, wf2_ref, bf2_ref,
                out_ref, acc_ref):
    f = pl.program_id(1)
    x = x_ref[...]
    h = jnp.dot(x.astype(jnp.bfloat16), wf1_ref[...],
                preferred_element_type=jnp.float32) + bf1_ref[...]
    h = _gelu_exact(h)
    p = jnp.dot(h.astype(jnp.bfloat16), wf2_ref[...],
                preferred_element_type=jnp.float32)

    @pl.when(f == 0)
    def _():
        acc_ref[...] = p

    @pl.when(f > 0)
    def _():
        acc_ref[...] += p

    @pl.when(f == _NF - 1)
    def _():
        out_ref[...] = x + w_ref[...] * (acc_ref[...] + bf2_ref[...])


def kernel(hidden_states, W1, b1, W2, b2, router_weight, router_bias,
           Wf1, bf1, Wf2, bf2):
    w, _k = pl.pallas_call(
        _head_kernel,
        out_shape=[
            jax.ShapeDtypeStruct((B, S), jnp.float32),
            jax.ShapeDtypeStruct((1, 1), jnp.int32),
        ],
    )(hidden_states, W1, b1.reshape(1, D // 4), W2, b2.reshape(1, 1),
      router_weight.reshape(1, D), router_bias.reshape(1, 1))

    out = pl.pallas_call(
        _ffn_kernel,
        grid=(_NT, _NF),
        in_specs=[
            pl.BlockSpec((_T, D), lambda t, f: (t, 0)),
            pl.BlockSpec((_T, 1), lambda t, f: (t, 0)),
            pl.BlockSpec((D, _F), lambda t, f: (0, f)),
            pl.BlockSpec((1, _F), lambda t, f: (0, f)),
            pl.BlockSpec((_F, D), lambda t, f: (f, 0)),
            pl.BlockSpec((1, D), lambda t, f: (0, 0)),
        ],
        out_specs=pl.BlockSpec((_T, D), lambda t, f: (t, 0)),
        out_shape=jax.ShapeDtypeStruct((BS, D), jnp.float32),
        scratch_shapes=[pltpu.VMEM((_T, D), jnp.float32)],
        compiler_params=pltpu.CompilerParams(
            dimension_semantics=("arbitrary", "arbitrary")),
    )(hidden_states.reshape(BS, D), w.reshape(BS, 1),
      Wf1.astype(jnp.bfloat16), bf1.reshape(1, DFF),
      Wf2.astype(jnp.bfloat16), bf2.reshape(1, D))

    return out.reshape(B, S, D)
